# baseline (speedup 1.0000x reference)
import functools

import jax
import jax.numpy as jnp
from jax import lax
from jax.experimental import pallas as pl
from jax.experimental.pallas import tpu as pltpu
from jax.experimental.pallas import tpu_sc as plsc

VOCAB = 1000000
EMBED = 128
BATCH = 16384

_NC = 2
_NS = 16
_NW = _NC * _NS
_CH = 128
_NSPLIT = 2


def _sc_gather_body(nch, idx_a_hbm, idx_b_hbm, table_hbm, out_a, out_b,
                    idx_v, rows_v, gsem0, gsem1, ssem0, ssem1):
    bpw = nch * _CH
    wid = lax.axis_index("s") * _NC + lax.axis_index("c")
    base = wid * bpw
    ia = pltpu.async_copy(idx_a_hbm.at[pl.ds(base, bpw)], idx_v.at[0], gsem0)
    ib = pltpu.async_copy(idx_b_hbm.at[pl.ds(base, bpw)], idx_v.at[1], gsem1)
    ia.wait()
    ib.wait()

    chunks = [(l, c) for l in range(2) for c in range(nch)]
    outs = (out_a, out_b)
    gsems = (gsem0, gsem1)
    ssems = (ssem0, ssem1)
    n = len(chunks)
    gathers = [None] * n
    stores = [None] * n
    for k in range(n + 1):
        if k >= 2:
            stores[k - 2].wait()
        if k < n:
            l, c = chunks[k]
            gathers[k] = pltpu.async_copy(
                table_hbm.at[idx_v.at[l, pl.ds(c * _CH, _CH)]],
                rows_v.at[k % 2],
                gsems[k % 2])
        if k >= 1:
            l, c = chunks[k - 1]
            gathers[k - 1].wait()
            stores[k - 1] = pltpu.async_copy(
                rows_v.at[(k - 1) % 2],
                outs[l].at[pl.ds(base + c * _CH, _CH)],
                ssems[(k - 1) % 2])
    stores[n - 1].wait()


def _sc_gather(idx_a, idx_b, emb):
    nrows = idx_a.shape[0]
    bpw = nrows // _NW
    nch = bpw // _CH
    mesh = plsc.VectorSubcoreMesh(core_axis_name="c", subcore_axis_name="s")
    fn = functools.partial(
        pl.kernel,
        mesh=mesh,
        out_type=[
            jax.ShapeDtypeStruct((nrows, EMBED), jnp.float32),
            jax.ShapeDtypeStruct((nrows, EMBED), jnp.float32),
        ],
        scratch_types=[
            pltpu.VMEM((2, bpw), jnp.int32),
            pltpu.VMEM((2, _CH, EMBED), jnp.float32),
            pltpu.SemaphoreType.DMA,
            pltpu.SemaphoreType.DMA,
            pltpu.SemaphoreType.DMA,
            pltpu.SemaphoreType.DMA,
        ],
    )(functools.partial(_sc_gather_body, nch))
    return fn(idx_a, idx_b, emb)


_BB = 4096


def _mlp_body(ea_ref, eb_ref, w1_ref, b1_ref, w2_ref, b2_ref, out_ref):
    ea = ea_ref[...]
    eb = eb_ref[...]
    h = lax.dot_general(ea, w1_ref[:, :EMBED], (((1,), (1,)), ((), ())),
                        preferred_element_type=jnp.float32)
    h = h + lax.dot_general(eb, w1_ref[:, EMBED:], (((1,), (1,)), ((), ())),
                            preferred_element_type=jnp.float32)
    h = h + b1_ref[...]
    h = jnp.maximum(h, 0.0)
    out = lax.dot_general(w2_ref[...], h, (((1,), (1,)), ((), ())),
                          preferred_element_type=jnp.float32)
    out_ref[0, 0, :] = out[0, :] + b2_ref[0]


def _mlp(ea, eb, W1, b1, W2, b2):
    nrows = ea.shape[0]
    nb = nrows // _BB
    out2d = pl.pallas_call(
        _mlp_body,
        grid=(nb,),
        in_specs=[
            pl.BlockSpec((_BB, EMBED), lambda i: (i, 0)),
            pl.BlockSpec((_BB, EMBED), lambda i: (i, 0)),
            pl.BlockSpec((EMBED, 2 * EMBED), lambda i: (0, 0)),
            pl.BlockSpec((1, EMBED), lambda i: (0, 0)),
            pl.BlockSpec((1, EMBED), lambda i: (0, 0)),
            pl.BlockSpec(memory_space=pltpu.SMEM),
        ],
        out_specs=pl.BlockSpec((1, 1, _BB), lambda i: (i, 0, 0)),
        out_shape=jax.ShapeDtypeStruct((nb, 1, _BB), jnp.float32),
    )(ea, eb, W1, b1.reshape(1, EMBED), W2, b2)
    return out2d.reshape(nrows)


def kernel(idx_a, idx_b, emb, W1, b1, W2, b2):
    piece = BATCH // _NSPLIT
    outs = []
    for s in range(_NSPLIT):
        sl = slice(s * piece, (s + 1) * piece)
        ea, eb = _sc_gather(idx_a[sl], idx_b[sl], emb)
        outs.append(_mlp(ea, eb, W1, b1, W2, b2))
    return jnp.concatenate(outs)


# speedup vs baseline: 1.1202x; 1.1202x over previous
import functools

import jax
import jax.numpy as jnp
from jax import lax
from jax.experimental import pallas as pl
from jax.experimental.pallas import tpu as pltpu
from jax.experimental.pallas import tpu_sc as plsc

VOCAB = 1000000
EMBED = 128
BATCH = 16384

_NC = 2
_NS = 16
_NW = _NC * _NS
_CH = 128
_NSPLIT = 2


def _sc_gather_body(nch, idx_a_hbm, idx_b_hbm, table_hbm, out_a, out_b,
                    idx_v, rows_v, gsem0, gsem1, ssem0, ssem1):
    bpw = nch * _CH
    wid = lax.axis_index("s") * _NC + lax.axis_index("c")
    base = wid * bpw
    ia = pltpu.async_copy(idx_a_hbm.at[pl.ds(base, bpw)], idx_v.at[0], gsem0)
    ib = pltpu.async_copy(idx_b_hbm.at[pl.ds(base, bpw)], idx_v.at[1], gsem1)
    ia.wait()
    ib.wait()

    chunks = [(l, c) for l in range(2) for c in range(nch)]
    outs = (out_a, out_b)
    gsems = (gsem0, gsem1)
    ssems = (ssem0, ssem1)
    n = len(chunks)
    gathers = [None] * n
    stores = [None] * n
    for k in range(n + 1):
        if k >= 2:
            stores[k - 2].wait()
        if k < n:
            l, c = chunks[k]
            gathers[k] = pltpu.async_copy(
                table_hbm.at[idx_v.at[l, pl.ds(c * _CH, _CH)]],
                rows_v.at[k % 2],
                gsems[k % 2])
        if k >= 1:
            l, c = chunks[k - 1]
            gathers[k - 1].wait()
            stores[k - 1] = pltpu.async_copy(
                rows_v.at[(k - 1) % 2],
                outs[l].at[pl.ds(base + c * _CH, _CH)],
                ssems[(k - 1) % 2])
    stores[n - 1].wait()


def _sc_gather(idx_a, idx_b, emb):
    nrows = idx_a.shape[0]
    bpw = nrows // _NW
    nch = bpw // _CH
    mesh = plsc.VectorSubcoreMesh(core_axis_name="c", subcore_axis_name="s")
    fn = functools.partial(
        pl.kernel,
        mesh=mesh,
        out_type=[
            jax.ShapeDtypeStruct((nrows, EMBED), jnp.float32),
            jax.ShapeDtypeStruct((nrows, EMBED), jnp.float32),
        ],
        scratch_types=[
            pltpu.VMEM((2, bpw), jnp.int32),
            pltpu.VMEM((2, _CH, EMBED), jnp.float32),
            pltpu.SemaphoreType.DMA,
            pltpu.SemaphoreType.DMA,
            pltpu.SemaphoreType.DMA,
            pltpu.SemaphoreType.DMA,
        ],
    )(functools.partial(_sc_gather_body, nch))
    return fn(idx_a, idx_b, emb)


_BB = 8192


def _mlp_body(ea_ref, eb_ref, w1_ref, b1_ref, w2_ref, b2_ref, out_ref):
    ea = ea_ref[...]
    eb = eb_ref[...]
    h = lax.dot_general(ea, w1_ref[:, :EMBED], (((1,), (1,)), ((), ())),
                        preferred_element_type=jnp.float32)
    h = h + lax.dot_general(eb, w1_ref[:, EMBED:], (((1,), (1,)), ((), ())),
                            preferred_element_type=jnp.float32)
    h = h + b1_ref[...]
    h = jnp.maximum(h, 0.0)
    out = lax.dot_general(w2_ref[...], h, (((1,), (1,)), ((), ())),
                          preferred_element_type=jnp.float32)
    out_ref[0, 0, :] = out[0, :] + b2_ref[0]


def _mlp(ea, eb, W1, b1, W2, b2):
    nrows = ea.shape[0]
    nb = nrows // _BB
    out2d = pl.pallas_call(
        _mlp_body,
        grid=(nb,),
        in_specs=[
            pl.BlockSpec((_BB, EMBED), lambda i: (i, 0)),
            pl.BlockSpec((_BB, EMBED), lambda i: (i, 0)),
            pl.BlockSpec((EMBED, 2 * EMBED), lambda i: (0, 0)),
            pl.BlockSpec((1, EMBED), lambda i: (0, 0)),
            pl.BlockSpec((1, EMBED), lambda i: (0, 0)),
            pl.BlockSpec(memory_space=pltpu.SMEM),
        ],
        out_specs=pl.BlockSpec((1, 1, _BB), lambda i: (i, 0, 0)),
        out_shape=jax.ShapeDtypeStruct((nb, 1, _BB), jnp.float32),
    )(ea, eb, W1, b1.reshape(1, EMBED), W2, b2)
    return out2d.reshape(nrows)


def kernel(idx_a, idx_b, emb, W1, b1, W2, b2):
    ea, eb = _sc_gather(idx_a, idx_b, emb)
    return _mlp(ea, eb, W1, b1, W2, b2)
